# baseline (speedup 1.0000x reference)
import functools

import jax
import jax.numpy as jnp
from jax import lax
from jax.experimental import pallas as pl
from jax.experimental.pallas import tpu as pltpu
from jax.experimental.pallas import tpu_sc as plsc

VOCAB = 100000
D = 128
L = 16384

NC = 2
NS = 16
NW = NC * NS
IDS_PER_W = L // NW
CHUNK = 128
NCHUNK = IDS_PER_W // CHUNK
NLANE = 16
NVREG = D // NLANE

BLK = 16384
NBLK_FRONT = 4
SC_NBLK = 2
TAIL_BLK = NBLK_FRONT + SC_NBLK
V_FRONT = NBLK_FRONT * BLK
V_SC = SC_NBLK * BLK
V_TAIL = VOCAB - V_FRONT - V_SC

ROWS_PER_SUB = V_SC // NW
WTILE = 128
NTILES = ROWS_PER_SUB // WTILE
JUNROLL = 16

_sc_mesh = plsc.VectorSubcoreMesh(core_axis_name="c", subcore_axis_name="s")

UNROLL = 4


@functools.partial(
    pl.kernel,
    mesh=_sc_mesh,
    out_type=jax.ShapeDtypeStruct((NW, D), jnp.float32),
    scratch_types=[
        pltpu.VMEM((NCHUNK, CHUNK), jnp.int32),
        pltpu.VMEM((2, CHUNK, D), jnp.float32),
        pltpu.VMEM((D,), jnp.float32),
        pltpu.SemaphoreType.DMA,
        pltpu.SemaphoreType.DMA,
    ],
)
def _gather_sum(ids_hbm, emb_hbm, out_hbm, idx_v, rows_v, out_v, sem0, sem1):
    sems = (sem0, sem1)
    wid = lax.axis_index("s") * NC + lax.axis_index("c")
    pltpu.sync_copy(ids_hbm.at[wid], idx_v)
    copies = [pltpu.async_copy(emb_hbm.at[idx_v.at[0]], rows_v.at[0], sems[0])]
    acc = tuple(jnp.zeros((NLANE,), jnp.float32) for _ in range(NVREG))
    for k in range(NCHUNK):
        if k + 1 < NCHUNK:
            copies.append(
                pltpu.async_copy(
                    emb_hbm.at[idx_v.at[k + 1]], rows_v.at[(k + 1) % 2],
                    sems[(k + 1) % 2],
                )
            )
        copies[k].wait()
        buf = rows_v.at[k % 2]

        def body(i, carry):
            for u in range(UNROLL):
                carry = tuple(
                    carry[j] + buf[i * UNROLL + u, pl.ds(j * NLANE, NLANE)]
                    for j in range(NVREG)
                )
            return carry

        acc = lax.fori_loop(0, CHUNK // UNROLL, body, acc)
    for j in range(NVREG):
        out_v[pl.ds(j * NLANE, NLANE)] = acc[j]
    pltpu.sync_copy(out_v, out_hbm.at[wid])


@functools.partial(
    pl.kernel,
    mesh=_sc_mesh,
    out_type=jax.ShapeDtypeStruct((V_SC,), jnp.float32),
    scratch_types=[
        pltpu.VMEM((NW, D), jnp.float32),
        pltpu.VMEM((WTILE * D,), jnp.float32),
        pltpu.VMEM((WTILE * D,), jnp.float32),
        pltpu.VMEM((ROWS_PER_SUB,), jnp.float32),
        pltpu.VMEM((ROWS_PER_SUB,), jnp.float32),
        pltpu.SemaphoreType.DMA,
        pltpu.SemaphoreType.DMA,
        pltpu.SemaphoreType.DMA,
    ],
)
def _sc_matvec(parts_hbm, w_hbm, b_hbm, out_hbm,
               parts_v, wtile0_v, wtile1_v, b_v, out_v, sem0, sem1, semb):
    wbufs = (wtile0_v, wtile1_v)
    wid = lax.axis_index("s") * NC + lax.axis_index("c")
    row0 = V_FRONT + wid * ROWS_PER_SUB

    sems = (sem0, sem1)
    bcopy = pltpu.async_copy(b_hbm.at[pl.ds(row0, ROWS_PER_SUB)], b_v, semb)
    copies = [pltpu.async_copy(
        w_hbm.at[pl.ds(row0 * D, WTILE * D)], wbufs[0], sems[0])]
    pltpu.sync_copy(parts_hbm, parts_v)

    def rbody(i, carry):
        return tuple(
            carry[j] + parts_v[i, pl.ds(j * NLANE, NLANE)]
            for j in range(NVREG)
        )

    svecs = lax.fori_loop(0, NW, rbody, tuple(
        jnp.zeros((NLANE,), jnp.float32) for _ in range(NVREG)))
    bcopy.wait()

    lanes = lax.iota(jnp.int32, NLANE)
    masks = [lanes == r for r in range(NLANE)]
    zvec = jnp.zeros((NLANE,), jnp.float32)
    for t in range(NTILES):
        if t + 1 < NTILES:
            copies.append(
                pltpu.async_copy(
                    w_hbm.at[pl.ds((row0 + (t + 1) * WTILE) * D, WTILE * D)],
                    wbufs[(t + 1) % 2], sems[(t + 1) % 2],
                )
            )
        copies[t].wait()
        wtile = wbufs[t % 2]

        def gbody(g, _):
            out_acc = b_v[pl.ds(t * WTILE + g * NLANE, NLANE)]
            for r in range(NLANE):
                off = (g * NLANE + r) * D
                rowacc = wtile[pl.ds(off, NLANE)] * svecs[0]
                for c in range(1, NVREG):
                    rowacc = rowacc + (
                        wtile[pl.ds(off + c * NLANE, NLANE)] * svecs[c])
                for sh in (8, 4, 2, 1):
                    rowacc = rowacc + rowacc.at[lanes ^ sh].get(
                        mode="promise_in_bounds")
                out_acc = out_acc + jnp.where(masks[r], rowacc, zvec)
            out_v[pl.ds(t * WTILE + g * NLANE, NLANE)] = out_acc
            return 0

        lax.fori_loop(0, WTILE // NLANE, gbody, 0)
    pltpu.sync_copy(out_v, out_hbm.at[pl.ds(wid * ROWS_PER_SUB, ROWS_PER_SUB)])


def _matvec_body(p_ref, w_ref, b_ref, bt_ref, of_ref, ot_ref):
    i = pl.program_id(0)
    s = jnp.sum(p_ref[...], axis=0, keepdims=True)
    mv = lax.dot_general(
        s, w_ref[...], (((1,), (1,)), ((), ())),
        preferred_element_type=jnp.float32,
    )

    @pl.when(i < NBLK_FRONT)
    def _():
        of_ref[...] = mv + b_ref[...]

    @pl.when(i == NBLK_FRONT)
    def _():
        ot_ref[...] = mv[:, :V_TAIL] + bt_ref[...]


def kernel(context_ids, embedding, W, b):
    ids3 = context_ids.reshape(NW, NCHUNK, CHUNK)
    partials = _gather_sum(ids3, embedding)
    out_sc = _sc_matvec(partials, W.reshape(VOCAB * D), b)
    b_tail = b[V_FRONT + V_SC:].reshape(1, V_TAIL)
    out_front, out_tail = pl.pallas_call(
        _matvec_body,
        grid=(NBLK_FRONT + 1,),
        in_specs=[
            pl.BlockSpec((NW, D), lambda i: (0, 0)),
            pl.BlockSpec(
                (BLK, D), lambda i: (jnp.where(i < NBLK_FRONT, i, TAIL_BLK), 0)),
            pl.BlockSpec(
                (1, BLK),
                lambda i: (0, jnp.where(i < NBLK_FRONT, i, NBLK_FRONT - 1))),
            pl.BlockSpec((1, V_TAIL), lambda i: (0, 0)),
        ],
        out_specs=[
            pl.BlockSpec(
                (1, BLK),
                lambda i: (0, jnp.where(i < NBLK_FRONT, i, NBLK_FRONT - 1))),
            pl.BlockSpec((1, V_TAIL), lambda i: (0, 0)),
        ],
        out_shape=[
            jax.ShapeDtypeStruct((1, V_FRONT), jnp.float32),
            jax.ShapeDtypeStruct((1, V_TAIL), jnp.float32),
        ],
    )(partials, W, b.reshape(1, VOCAB), b_tail)
    return jnp.concatenate(
        [out_front, out_sc.reshape(1, V_SC), out_tail], axis=1)


# speedup vs baseline: 1.1764x; 1.1764x over previous
import functools

import jax
import jax.numpy as jnp
from jax import lax
from jax.experimental import pallas as pl
from jax.experimental.pallas import tpu as pltpu
from jax.experimental.pallas import tpu_sc as plsc

VOCAB = 100000
D = 128
L = 16384

NC = 2
NS = 16
NW = NC * NS
IDS_PER_W = L // NW
CHUNK = 128
NCHUNK = IDS_PER_W // CHUNK
NLANE = 16
NVREG = D // NLANE

BLK = 16384
NBLK_FRONT = 4
SC_NBLK = 2
TAIL_BLK = NBLK_FRONT + SC_NBLK
V_FRONT = NBLK_FRONT * BLK
V_SC = SC_NBLK * BLK
V_TAIL = VOCAB - V_FRONT - V_SC

ROWS_PER_SUB = V_SC // NW
WTILE = 128
NTILES = ROWS_PER_SUB // WTILE
JUNROLL = 16

_sc_mesh = plsc.VectorSubcoreMesh(core_axis_name="c", subcore_axis_name="s")

UNROLL = 4


@functools.partial(
    pl.kernel,
    mesh=_sc_mesh,
    out_type=jax.ShapeDtypeStruct((NW, D), jnp.float32),
    scratch_types=[
        pltpu.VMEM((NCHUNK, CHUNK), jnp.int32),
        pltpu.VMEM((NCHUNK, CHUNK, D), jnp.float32),
        pltpu.VMEM((D,), jnp.float32),
        pltpu.SemaphoreType.DMA,
        pltpu.SemaphoreType.DMA,
        pltpu.SemaphoreType.DMA,
        pltpu.SemaphoreType.DMA,
    ],
)
def _gather_sum(ids_hbm, emb_hbm, out_hbm, idx_v, rows_v, out_v,
                sem0, sem1, sem2, sem3):
    sems = (sem0, sem1, sem2, sem3)
    wid = lax.axis_index("s") * NC + lax.axis_index("c")
    pltpu.sync_copy(ids_hbm.at[wid], idx_v)
    copies = [
        pltpu.async_copy(emb_hbm.at[idx_v.at[k]], rows_v.at[k], sems[k])
        for k in range(NCHUNK)
    ]
    acc = tuple(jnp.zeros((NLANE,), jnp.float32) for _ in range(NVREG))
    for k in range(NCHUNK):
        copies[k].wait()
        buf = rows_v.at[k]

        def body(i, carry):
            for u in range(UNROLL):
                carry = tuple(
                    carry[j] + buf[i * UNROLL + u, pl.ds(j * NLANE, NLANE)]
                    for j in range(NVREG)
                )
            return carry

        acc = lax.fori_loop(0, CHUNK // UNROLL, body, acc)
    for j in range(NVREG):
        out_v[pl.ds(j * NLANE, NLANE)] = acc[j]
    pltpu.sync_copy(out_v, out_hbm.at[wid])


def _matvec_body(p_ref, w_ref, b_ref, o_ref):
    s = jnp.sum(p_ref[...], axis=0, keepdims=True)
    o_ref[...] = (
        lax.dot_general(
            s, w_ref[...], (((1,), (1,)), ((), ())),
            preferred_element_type=jnp.float32,
        )
        + b_ref[...]
    )


def kernel(context_ids, embedding, W, b):
    ids3 = context_ids.reshape(NW, NCHUNK, CHUNK)
    partials = _gather_sum(ids3, embedding)
    out = pl.pallas_call(
        _matvec_body,
        grid=(pl.cdiv(VOCAB, BLK),),
        in_specs=[
            pl.BlockSpec((NW, D), lambda i: (0, 0)),
            pl.BlockSpec((BLK, D), lambda i: (i, 0)),
            pl.BlockSpec((1, BLK), lambda i: (0, i)),
        ],
        out_specs=pl.BlockSpec((1, BLK), lambda i: (0, i)),
        out_shape=jax.ShapeDtypeStruct((1, VOCAB), jnp.float32),
    )(partials, W, b.reshape(1, VOCAB))
    return out
